# baseline (speedup 1.0000x reference)
import jax
import jax.numpy as jnp
import numpy as np
from jax.experimental import pallas as pl
from jax.experimental.pallas import tpu as pltpu
try:
    from jax.experimental.shard_map import shard_map
except ImportError:
    from jax.shard_map import shard_map

B, L, E, FEAT, D_IN, D_H, D_OUT = 16384, 8, 8, 32, 64, 128, 32


def _fused_kernel(feat_ref, p_ref, w0_ref, b0_ref, w1_ref, b1_ref, ids_ref,
                  out_ref):
    bb = feat_ref.shape[0]
    p = p_ref[...]
    feats = feat_ref[...]
    ids = ids_ref[...]
    w0 = w0_ref[...]
    b0 = b0_ref[...]
    w1 = w1_ref[...]
    b1 = b1_ref[...]

    def tree_select(parts, masks):
        while len(parts) > 1:
            m = masks[0]
            parts = [jnp.where(m, hi, lo)
                     for lo, hi in zip(parts[0::2], parts[1::2])]
            masks = masks[1:]
        return parts[0]

    for n in range(L):
        idn = ids[:, n:n + 1]
        masks = [(idn & 1) != 0, (idn & 2) != 0, (idn & 4) != 0]
        x = jnp.concatenate([p, feats[:, n * FEAT:(n + 1) * FEAT]], axis=1)
        pre = jnp.dot(x.astype(jnp.bfloat16), w0,
                      preferred_element_type=jnp.float32) + b0
        psel = tree_select(
            [pre[:, i * D_H:(i + 1) * D_H] for i in range(E)], masks)
        h = jnp.tanh(psel).astype(jnp.bfloat16)
        os = [jnp.dot(h, w1[i * D_H:(i + 1) * D_H],
                      preferred_element_type=jnp.float32) + b1[i]
              for i in range(E)]
        p = tree_select(os, masks)
    out_ref[...] = jnp.maximum(p, 0.0)


def _run_block(feats, p_in, w0cat, b0cat, w1cat, b1, ids):
    nrows = feats.shape[0]
    BB = 1024
    grid = (nrows // BB,)
    return pl.pallas_call(
        _fused_kernel,
        grid=grid,
        in_specs=[
            pl.BlockSpec((BB, L * FEAT), lambda b: (b, 0)),
            pl.BlockSpec((BB, D_OUT), lambda b: (b, 0)),
            pl.BlockSpec((D_IN, E * D_H), lambda b: (0, 0)),
            pl.BlockSpec((1, E * D_H), lambda b: (0, 0)),
            pl.BlockSpec((E * D_H, D_OUT), lambda b: (0, 0)),
            pl.BlockSpec((E, D_OUT), lambda b: (0, 0)),
            pl.BlockSpec((BB, L), lambda b: (b, 0)),
        ],
        out_specs=pl.BlockSpec((BB, D_OUT), lambda b: (b, 0)),
        out_shape=jax.ShapeDtypeStruct((nrows, D_OUT), jnp.float32),
        compiler_params=pltpu.CompilerParams(
            dimension_semantics=("parallel",)),
    )(feats, p_in, w0cat, b0cat, w1cat, b1, ids)


def kernel(mod_feat_seq, p_in, W0, b0, W1, b1, mod_id_seq):
    w0cat = jnp.transpose(W0, (2, 0, 1)).reshape(D_IN, E * D_H)
    w0cat = w0cat.astype(jnp.bfloat16)
    b0cat = b0.reshape(1, E * D_H)
    w1cat = jnp.transpose(W1, (0, 2, 1)).reshape(E * D_H, D_OUT)
    w1cat = w1cat.astype(jnp.bfloat16)

    feats = mod_feat_seq.reshape(B, L * FEAT)
    ids = mod_id_seq.astype(jnp.int32)

    devs = jax.devices()
    ndev = len(devs) if B % (len(devs) * 1024) == 0 else 1
    if ndev > 1:
        mesh = jax.sharding.Mesh(np.asarray(devs), ("d",))
        rep = jax.sharding.PartitionSpec()
        row = jax.sharding.PartitionSpec("d")
        fn = shard_map(
            _run_block, mesh=mesh,
            in_specs=(row, row, rep, rep, rep, rep, row),
            out_specs=row, check_rep=False)
        return fn(feats, p_in, w0cat, b0cat, w1cat, b1, ids)
    return _run_block(feats, p_in, w0cat, b0cat, w1cat, b1, ids)


# speedup vs baseline: 3.4827x; 3.4827x over previous
import jax
import jax.numpy as jnp
from jax.experimental import pallas as pl
from jax.experimental.pallas import tpu as pltpu

B, L, E, FEAT, D_IN, D_H, D_OUT = 16384, 8, 8, 32, 64, 128, 32


def _fused_kernel(feat_ref, p_ref, w0_ref, b0_ref, w1_ref, b1_ref, ids_ref,
                  out_ref):
    bb = feat_ref.shape[0]
    p = p_ref[...]
    feats = feat_ref[...]
    ids = ids_ref[...]
    w0 = w0_ref[...]
    b0 = b0_ref[...]
    w1 = w1_ref[...]
    b1 = b1_ref[...]

    def tree_select(parts, masks):
        while len(parts) > 1:
            m = masks[0]
            parts = [jnp.where(m, hi, lo)
                     for lo, hi in zip(parts[0::2], parts[1::2])]
            masks = masks[1:]
        return parts[0]

    for n in range(L):
        idn = ids[:, n:n + 1]
        masks = [(idn & 1) != 0, (idn & 2) != 0, (idn & 4) != 0]
        x = jnp.concatenate([p, feats[:, n * FEAT:(n + 1) * FEAT]], axis=1)
        pre = jnp.dot(x.astype(jnp.bfloat16), w0,
                      preferred_element_type=jnp.float32) + b0
        psel = tree_select(
            [pre[:, i * D_H:(i + 1) * D_H] for i in range(E)], masks)
        h = jnp.tanh(psel).astype(jnp.bfloat16)
        os = [jnp.dot(h, w1[i * D_H:(i + 1) * D_H],
                      preferred_element_type=jnp.float32) + b1[i]
              for i in range(E)]
        p = tree_select(os, masks)
    out_ref[...] = jnp.maximum(p, 0.0)


def kernel(mod_feat_seq, p_in, W0, b0, W1, b1, mod_id_seq):
    w0cat = jnp.transpose(W0, (2, 0, 1)).reshape(D_IN, E * D_H)
    w0cat = w0cat.astype(jnp.bfloat16)
    b0cat = b0.reshape(1, E * D_H)
    w1cat = jnp.transpose(W1, (0, 2, 1)).reshape(E * D_H, D_OUT)
    w1cat = w1cat.astype(jnp.bfloat16)

    feats = mod_feat_seq.reshape(B, L * FEAT)
    ids = mod_id_seq.astype(jnp.int32)

    BB = 1024
    grid = (B // BB,)
    return pl.pallas_call(
        _fused_kernel,
        grid=grid,
        in_specs=[
            pl.BlockSpec((BB, L * FEAT), lambda b: (b, 0)),
            pl.BlockSpec((BB, D_OUT), lambda b: (b, 0)),
            pl.BlockSpec((D_IN, E * D_H), lambda b: (0, 0)),
            pl.BlockSpec((1, E * D_H), lambda b: (0, 0)),
            pl.BlockSpec((E * D_H, D_OUT), lambda b: (0, 0)),
            pl.BlockSpec((E, D_OUT), lambda b: (0, 0)),
            pl.BlockSpec((BB, L), lambda b: (b, 0)),
        ],
        out_specs=pl.BlockSpec((BB, D_OUT), lambda b: (b, 0)),
        out_shape=jax.ShapeDtypeStruct((B, D_OUT), jnp.float32),
        compiler_params=pltpu.CompilerParams(
            dimension_semantics=("parallel",)),
    )(feats, p_in, w0cat, b0cat, w1cat, b1, ids)
